# baseline (speedup 1.0000x reference)
import functools

import jax
import jax.numpy as jnp
from jax import lax
from jax.experimental import pallas as pl
from jax.experimental.pallas import tpu as pltpu
from jax.experimental.pallas import tpu_sc as plsc

INPUT_DIM = 1000000
HALF = INPUT_DIM // 2
D = 32

NC = 2
NS = 16
NW = NC * NS
L = 16

CHUNK = 1024
BLK = 128
NBLK = CHUNK // BLK


def _embed_kernel(B: int):
    n_per_w = B // NW
    n_chunks = n_per_w // CHUNK

    mesh = plsc.VectorSubcoreMesh(core_axis_name="c", subcore_axis_name="s")

    @functools.partial(
        pl.kernel,
        out_type=jax.ShapeDtypeStruct((B, D), jnp.float32),
        mesh=mesh,
        compiler_params=pltpu.CompilerParams(use_tc_tiling_on_sc=False),
        scratch_types=[
            pltpu.VMEM((CHUNK,), jnp.int32),
            pltpu.VMEM((CHUNK,), jnp.int32),
            pltpu.VMEM((CHUNK,), jnp.int32),
            pltpu.VMEM((NBLK, BLK), jnp.int32),
            pltpu.VMEM((NBLK, BLK), jnp.int32),
            pltpu.VMEM((CHUNK, D), jnp.float32),
            pltpu.VMEM((CHUNK, D), jnp.float32),
            pltpu.SemaphoreType.DMA,
            pltpu.SemaphoreType.DMA,
        ],
    )
    def k(idx_hbm, e1_hbm, e2_hbm, out_hbm,
          idx_v, idx1_v, idx2_v, pos1_v, pos2_v, rows1_v, rows2_v,
          gsem, ssem):
        wid = lax.axis_index("s") * NC + lax.axis_index("c")
        wbase = wid * n_per_w
        dump = wbase
        iota = lax.broadcasted_iota(jnp.int32, (L,), 0)

        def chunk_body(kk, carry):
            base = wbase + kk * CHUNK
            pltpu.sync_copy(idx_hbm.at[pl.ds(base, CHUNK)], idx_v)

            carry = jnp.where(kk == 0, idx_v[pl.ds(0, L)][0], carry)

            for g in range(CHUNK // L):
                idx16 = idx_v[pl.ds(g * L, L)]
                m = idx16 < HALF
                idx1_v[pl.ds(g * L, L)] = jnp.minimum(idx16, HALF - 1)
                idx2_v[pl.ds(g * L, L)] = jnp.maximum(idx16 - HALF, 0)
                pos = (base + g * L) + iota
                r, c = g // (BLK // L), (g % (BLK // L)) * L
                pos1_v[r, pl.ds(c, L)] = jnp.where(m, pos, dump)
                pos2_v[r, pl.ds(c, L)] = jnp.where(m, dump, pos)

            gathers = []
            for b in range(NBLK):
                s = pl.ds(b * BLK, BLK)
                gathers.append(pltpu.async_copy(
                    e1_hbm.at[idx1_v.at[s]], rows1_v.at[s], gsem))
                gathers.append(pltpu.async_copy(
                    e2_hbm.at[idx2_v.at[s]], rows2_v.at[s], gsem))
            for cp in gathers:
                cp.wait()

            scatters = []
            for b in range(NBLK):
                s = pl.ds(b * BLK, BLK)
                scatters.append(pltpu.async_copy(
                    rows1_v.at[s], out_hbm.at[pos1_v.at[b]], ssem))
                scatters.append(pltpu.async_copy(
                    rows2_v.at[s], out_hbm.at[pos2_v.at[b]], ssem))
            for cp in scatters:
                cp.wait()
            return carry

        iv = lax.fori_loop(0, n_chunks, chunk_body, 0)


        @pl.when(iv < HALF)
        def _():
            pltpu.sync_copy(e1_hbm.at[iv], rows1_v.at[0])

        @pl.when(iv >= HALF)
        def _():
            pltpu.sync_copy(e2_hbm.at[iv - HALF], rows1_v.at[0])

        pltpu.sync_copy(rows1_v.at[0], out_hbm.at[dump])

    return k


def kernel(inputs, e1, e2):
    bsz, hist = inputs.shape
    B = bsz * hist
    idx = inputs.reshape(B).astype(jnp.int32)
    out = _embed_kernel(B)(idx, e1, e2)
    return out.reshape(bsz, hist, D)


# speedup vs baseline: 3.3814x; 3.3814x over previous
import functools

import jax
import jax.numpy as jnp
from jax import lax
from jax.experimental import pallas as pl
from jax.experimental.pallas import tpu as pltpu
from jax.experimental.pallas import tpu_sc as plsc

INPUT_DIM = 1000000
HALF = INPUT_DIM // 2
D = 32

NC = 2
NS = 16
NW = NC * NS
L = 16

SEG = 12800
BLK = 128
CAP = SEG + 160


def _embed_kernel(B: int):
    n_per_w = B // NW
    n_seg = n_per_w // SEG

    mesh = plsc.VectorSubcoreMesh(core_axis_name="c", subcore_axis_name="s")

    @functools.partial(
        pl.kernel,
        out_type=jax.ShapeDtypeStruct((B, D), jnp.float32),
        mesh=mesh,
        compiler_params=pltpu.CompilerParams(use_tc_tiling_on_sc=False,
                                             needs_layout_passes=False),
        scratch_types=[
            pltpu.VMEM((SEG,), jnp.int32),
            pltpu.VMEM((CAP,), jnp.int32),
            pltpu.VMEM((CAP,), jnp.int32),
            pltpu.VMEM((CAP,), jnp.int32),
            pltpu.VMEM((CAP,), jnp.int32),
            pltpu.VMEM((2, BLK, D), jnp.float32),
            pltpu.VMEM((2, BLK), jnp.int32),
            pltpu.SemaphoreType.DMA,
            pltpu.SemaphoreType.DMA,
        ],
    )
    def k(idx_hbm, e1_hbm, e2_hbm, out_hbm,
          idx_v, idxb1, posb1, idxb2, posb2, rows_v, pstage, gsem, ssem):
        wid = lax.axis_index("s") * NC + lax.axis_index("c")
        wbase = wid * n_per_w
        iota = lax.broadcasted_iota(jnp.int32, (L,), 0)

        def run_table(idxb, posb, cnt, table_hbm):
            @pl.when(cnt > 0)
            def _():
                last = cnt - 1
                li = plsc.load_gather(idxb, [jnp.full((L,), last, jnp.int32)])
                lp = plsc.load_gather(posb, [jnp.full((L,), last, jnp.int32)])
                g0 = cnt - (cnt & (L - 1))
                keep = (g0 + iota) < cnt
                idxb[pl.ds(g0, L)] = jnp.where(keep, idxb[pl.ds(g0, L)], li)
                posb[pl.ds(g0, L)] = jnp.where(keep, posb[pl.ds(g0, L)], lp)
                for t in range(1, BLK // L + 1):
                    idxb[pl.ds(g0 + t * L, L)] = li
                    posb[pl.ds(g0 + t * L, L)] = lp

                nb = (cnt + BLK - 1) // BLK

                def fire_gather(b, slot):
                    return pltpu.async_copy(
                        table_hbm.at[idxb.at[pl.ds(b * BLK, BLK)]],
                        rows_v.at[slot], gsem)

                def stage_and_scatter(b, slot):
                    for t in range(BLK // L):
                        pstage[slot, pl.ds(t * L, L)] = (
                            posb[pl.ds(b * BLK + t * L, L)])
                    return pltpu.async_copy(
                        rows_v.at[slot], out_hbm.at[pstage.at[slot]], ssem)

                def body(p, carry):
                    b = p * 2
                    ga = fire_gather(b, 0)
                    ga.wait()
                    sa = stage_and_scatter(b, 0)

                    @pl.when(b + 1 < nb)
                    def _():
                        gb = fire_gather(b + 1, 1)
                        gb.wait()
                        sb = stage_and_scatter(b + 1, 1)
                        sb.wait()

                    sa.wait()
                    return carry

                lax.fori_loop(0, (nb + 1) // 2, body, 0)

        for seg in range(n_seg):
            seg_gbase = wbase + seg * SEG
            pltpu.sync_copy(idx_hbm.at[pl.ds(seg_gbase, SEG)], idx_v)

            def compact(g, carry):
                c1, c2 = carry
                idx16 = idx_v[pl.ds(g * L, L)]
                m = idx16 < HALF
                n1 = plsc.all_reduce_population_count(m)[0]
                pos16 = (seg_gbase + g * L) + iota
                plsc.store_compressed(idxb1.at[pl.ds(c1, L)], idx16, mask=m)
                plsc.store_compressed(posb1.at[pl.ds(c1, L)], pos16, mask=m)
                plsc.store_compressed(idxb2.at[pl.ds(c2, L)], idx16 - HALF,
                                      mask=~m)
                plsc.store_compressed(posb2.at[pl.ds(c2, L)], pos16, mask=~m)
                return (c1 + n1, c2 + (L - n1))

            zero = jnp.int32(0)
            c1, c2 = lax.fori_loop(0, SEG // L, compact, (zero, zero))

            run_table(idxb1, posb1, c1, e1_hbm)
            run_table(idxb2, posb2, c2, e2_hbm)

    return k


def kernel(inputs, e1, e2):
    bsz, hist = inputs.shape
    B = bsz * hist
    idx = inputs.reshape(B).astype(jnp.int32)
    out = _embed_kernel(B)(idx, e1, e2)
    return out.reshape(bsz, hist, D)
